# Initial kernel scaffold; baseline (speedup 1.0000x reference)
import jax
import jax.numpy as jnp
from jax.experimental import pallas as pl


def kernel(x, edge_index, gate_w, gate_b):
    raise NotImplementedError("write your pallas kernel here")


# speedup vs baseline: 3.2061x; 3.2061x over previous
import functools

import jax
import jax.numpy as jnp
from jax import lax
from jax.experimental import pallas as pl
from jax.experimental.pallas import tpu as pltpu
from jax.experimental.pallas import tpu_sc as plsc

N = 10000
D = 128
E = 320000
NC = 2
NS = 16
L = 16
NW = NC * NS
EPW = E // NW
C = 80
J = EPW // C
WCH = 80
NCH = N // WCH

_MESH = dict(core_axis_name="c", subcore_axis_name="s", num_cores=NC,
             num_subcores=NS)


def _scores_body(x_ref, w_ref, b_ref, o_ref):
  s = lax.dot_general(x_ref[...], w_ref[...], (((1,), (0,)), ((), ())),
                      preferred_element_type=jnp.float32)
  s = s + b_ref[0, 0]
  m = jnp.max(s)
  o_ref[...] = jnp.exp(s - m)


def _exp_scores(x, gate_w, gate_b):
  return pl.pallas_call(
      _scores_body,
      out_shape=jax.ShapeDtypeStruct((N, 1), jnp.float32),
  )(x, gate_w.reshape(D, 1), gate_b.reshape(1, 1))


def _own_body(src_hbm, dst_hbm, t_hbm, src_v, dst_v, keys_v, eids_v, sem):
  cid = lax.axis_index("c")
  sid = lax.axis_index("s")
  wid = sid * NC + cid
  pltpu.sync_copy(src_hbm.at[wid], src_v)
  pltpu.sync_copy(dst_hbm.at[wid], dst_v)
  iota = lax.iota(jnp.int32, L)

  @pl.loop(0, J)
  def _build(j):
    for i in range(C // L):
      sl = pl.ds(i * L, L)
      keys_v[j, sl] = dst_v[j, sl] * N + src_v[j, sl]
      eids_v[j, sl] = iota + (wid * EPW + j * C + i * L)

  @pl.loop(0, J)
  def _scatter(j):
    pltpu.async_copy(eids_v.at[j], t_hbm.at[keys_v.at[j]], sem).wait()


def _own_table(src_r, dst_r):
  kern = functools.partial(
      pl.kernel,
      out_type=jax.ShapeDtypeStruct((N * N,), jnp.int32),
      mesh=plsc.VectorSubcoreMesh(**_MESH),
      compiler_params=pltpu.CompilerParams(needs_layout_passes=False),
      scratch_types=[
          pltpu.VMEM((J, C), jnp.int32),
          pltpu.VMEM((J, C), jnp.int32),
          pltpu.VMEM((J, C), jnp.int32),
          pltpu.VMEM((J, C), jnp.int32),
          pltpu.SemaphoreType.DMA,
      ],
  )(_own_body)
  return kern(src_r, dst_r)


def _agg_body(edges_hbm, t_hbm, exps_hbm, x_hbm, a_hbm, d_hbm,
              e_c, keys_c, exps_v, rows_v, w_v, own_v, den_v,
              acc_sh, den_sh, sem_a, sem_b):
  cid = lax.axis_index("c")
  sid = lax.axis_index("s")
  wid = sid * NC + cid
  iota = lax.iota(jnp.int32, L)

  @pl.loop(0, C)
  def _zrow(r):
    for f in range(D // L):
      rows_v[r, pl.ds(f * L, L)] = jnp.zeros((L,), jnp.float32)

  @pl.loop(0, 1024 // L)
  def _zden_fill(i):
    den_v[pl.ds(i * L, L)] = jnp.zeros((L,), jnp.float32)

  for k in range(NCH // NS + 1):
    c = sid + k * NS

    @pl.when(c < NCH)
    def _zchunk():
      pltpu.sync_copy(rows_v, acc_sh.at[pl.ds(c * WCH, WCH), :])

  @pl.when(sid < 10)
  def _zden():
    pltpu.sync_copy(den_v.at[pl.ds(0, 1000)],
                    den_sh.at[pl.ds(sid * 1000, 1000)])

  pltpu.sync_copy(exps_hbm, exps_v)

  plsc.subcore_barrier()

  @pl.loop(0, J)
  def _chunk(j):
    pltpu.sync_copy(edges_hbm.at[wid, j], e_c)
    for i in range(C // L):
      sl = pl.ds(i * L, L)
      keys_c[sl] = e_c[0, sl] * N + e_c[1, sl]
    own_cp = pltpu.async_copy(t_hbm.at[keys_c], own_v, sem_a)
    rows_cp = pltpu.async_copy(x_hbm.at[e_c.at[1]], rows_v, sem_b)
    own_cp.wait()
    rows_cp.wait()
    for i in range(C // L):
      sl = pl.ds(i * L, L)
      e16 = plsc.load_gather(exps_v, [e_c[1, sl]])
      eid16 = iota + (wid * EPW + j * C + i * L)
      w_v[sl] = jnp.where(own_v[sl] == eid16, e16,
                          jnp.zeros((L,), jnp.float32))

    @pl.loop(0, C)
    def _scale(e):
      wb = plsc.load_gather(w_v, [jnp.full((L,), e, jnp.int32)])
      for f in range(D // L):
        fs = pl.ds(f * L, L)
        rows_v[e, fs] = rows_v[e, fs] * wb

    acc_cp = pltpu.async_copy(rows_v, acc_sh.at[e_c.at[0]], sem_a, add=True)
    den_cp = pltpu.async_copy(w_v, den_sh.at[e_c.at[0]], sem_b, add=True)
    acc_cp.wait()
    den_cp.wait()

  plsc.subcore_barrier()

  for k in range(NCH // NS + 1):
    c = sid + k * NS

    @pl.when(c < NCH)
    def _wchunk():
      pltpu.sync_copy(acc_sh.at[pl.ds(c * WCH, WCH), :], rows_v)
      pltpu.sync_copy(rows_v, a_hbm.at[cid, c])

  @pl.when(sid < 10)
  def _wden():
    pltpu.sync_copy(den_sh.at[pl.ds(sid * 1000, 1000)],
                    den_v.at[pl.ds(0, 1000)])
    pltpu.sync_copy(den_v, d_hbm.at[cid, sid])


def _aggregate(edges, table, exps, x):
  kern = functools.partial(
      pl.kernel,
      out_type=(jax.ShapeDtypeStruct((NC, NCH, WCH, D), jnp.float32),
                jax.ShapeDtypeStruct((NC, 10, 1024), jnp.float32)),
      mesh=plsc.VectorSubcoreMesh(**_MESH),
      compiler_params=pltpu.CompilerParams(needs_layout_passes=False),
      scratch_types=[
          pltpu.VMEM((2, C), jnp.int32),
          pltpu.VMEM((C,), jnp.int32),
          pltpu.VMEM((N,), jnp.float32),
          pltpu.VMEM((C, D), jnp.float32),
          pltpu.VMEM((C,), jnp.float32),
          pltpu.VMEM((C,), jnp.int32),
          pltpu.VMEM((1024,), jnp.float32),
          pltpu.VMEM_SHARED((N, D), jnp.float32),
          pltpu.VMEM_SHARED((N,), jnp.float32),
          pltpu.SemaphoreType.DMA,
          pltpu.SemaphoreType.DMA,
      ],
  )(_agg_body)
  return kern(edges, table, exps, x)


def _final_body(x_ref, a_ref, d0_ref, d1_ref, o_ref):
  den = jnp.maximum(d0_ref[...] + d1_ref[...], 1e-16)
  att = (a_ref[0] + a_ref[1]) / den
  o_ref[...] = jnp.concatenate([x_ref[...], att], axis=1)


def _finalize(x, a_part, d_part):
  rows = 1000
  d0 = d_part[0].reshape(N, 1)
  d1 = d_part[1].reshape(N, 1)
  return pl.pallas_call(
      _final_body,
      grid=(N // rows,),
      in_specs=[
          pl.BlockSpec((rows, D), lambda g: (g, 0)),
          pl.BlockSpec((NC, rows, D), lambda g: (0, g, 0)),
          pl.BlockSpec((rows, 1), lambda g: (g, 0)),
          pl.BlockSpec((rows, 1), lambda g: (g, 0)),
      ],
      out_specs=pl.BlockSpec((rows, 2 * D), lambda g: (g, 0)),
      out_shape=jax.ShapeDtypeStruct((N, 2 * D), jnp.float32),
  )(x, a_part, d0, d1)


def kernel(x, edge_index, gate_w, gate_b):
  dst = edge_index[0].astype(jnp.int32).reshape(NW, J, C)
  src = edge_index[1].astype(jnp.int32).reshape(NW, J, C)
  edges = jnp.stack([dst, src], axis=2)
  exps = _exp_scores(x, gate_w, gate_b)
  table = _own_table(src, dst)
  a_part, d_part = _aggregate(edges, table, exps.reshape(N), x)
  den = d_part[:, :, :1000].reshape(NC, N)
  return _finalize(x, a_part.reshape(NC, N, D), den)
